# baseline (speedup 1.0000x reference)
import functools

import jax
import jax.numpy as jnp
from jax import lax
from jax.experimental import pallas as pl
from jax.experimental.pallas import tpu as pltpu
from jax.experimental.pallas import tpu_sc as plsc

NC, NS = 2, 16
NW = NC * NS
C = 16
NB = 4
LOOK = NB - 2


def _sc_embed(level_ids_flat, token_ids_flat, emb, level_embed):
    n = token_ids_flat.shape[0]
    d = emb.shape[1]
    nlev = level_embed.shape[0]
    rpw = n // NW
    nchunk = rpw // C
    nround = nchunk // NB
    mesh = plsc.VectorSubcoreMesh(core_axis_name="c", subcore_axis_name="s",
                                  num_cores=NC, num_subcores=NS)

    @functools.partial(
        pl.kernel,
        out_type=jax.ShapeDtypeStruct((n, d), jnp.float32),
        mesh=mesh,
        compiler_params=pltpu.CompilerParams(needs_layout_passes=False),
        scratch_types=[
            pltpu.VMEM((rpw,), jnp.int32),
            pltpu.VMEM((rpw,), jnp.int32),
            pltpu.VMEM((NB, C, d), jnp.float32),
            pltpu.VMEM((nlev, d), jnp.float32),
            pltpu.VMEM_SHARED((nlev, d), jnp.float32),
            pltpu.SemaphoreType.DMA((NB,)),
            pltpu.SemaphoreType.DMA((NB,)),
        ],
    )
    def k(lvl_hbm, tok_hbm, emb_hbm, lev_hbm, out_hbm,
          tok_v, lvl_v, rows, lev_v, lev_sh, sem_g, sem_s):
        wid = lax.axis_index("s") * NC + lax.axis_index("c")
        wbase = wid * rpw

        pltpu.sync_copy(tok_hbm.at[pl.ds(wbase, rpw)], tok_v)
        pltpu.sync_copy(lvl_hbm.at[pl.ds(wbase, rpw)], lvl_v)
        pltpu.sync_copy(lev_hbm, lev_v)

        @pl.when(lax.axis_index("s") == 0)
        def _():
            pltpu.sync_copy(lev_v, lev_sh)

        plsc.subcore_barrier()

        def issue_gather(b, c):
            pltpu.async_copy(emb_hbm.at[tok_v.at[pl.ds(c * C, C)]],
                             rows.at[b], sem_g.at[b])

        def wait_gather(b):
            pltpu.make_async_copy(emb_hbm.at[tok_v.at[pl.ds(0, C)]],
                                  rows.at[b], sem_g.at[b]).wait()

        def issue_store(b, c):
            base = wbase + c * C
            pltpu.async_copy(rows.at[b], out_hbm.at[pl.ds(base, C)],
                             sem_s.at[b])

        def wait_store(b):
            pltpu.make_async_copy(rows.at[b], out_hbm.at[pl.ds(0, C)],
                                  sem_s.at[b]).wait()

        for c0 in range(LOOK):
            issue_gather(c0, c0)

        def round_body(g, carry):
            for b in range(NB):
                c = g * NB + b
                wait_gather(b)

                @plsc.parallel_loop(0, d, unroll=4)
                def _(p, _b=b, _c=c):
                    lvl_vec = lvl_v[pl.ds(_c * C, 16)]
                    row_iota = lax.iota(jnp.int32, 16)
                    p_vec = jnp.zeros((16,), jnp.int32) + p
                    vals = plsc.load_gather(lev_v, [lvl_vec, p_vec])
                    plsc.addupdate_scatter(rows.at[_b], [row_iota, p_vec],
                                           vals)

                issue_store(b, c)

                bn = (b + LOOK) % NB

                @pl.when(c + LOOK >= NB)
                def _():
                    wait_store(bn)

                @pl.when(c + LOOK < nchunk)
                def _():
                    issue_gather(bn, c + LOOK)
            return carry

        lax.fori_loop(0, nround, round_body, 0)
        for ct in range(nchunk - NB + LOOK, nchunk):
            wait_store(ct % NB)

    return k(level_ids_flat, token_ids_flat, emb, level_embed)


def kernel(level_ids, token_ids, emb, level_embed):
    b, s = token_ids.shape
    n = b * s
    out = _sc_embed(level_ids.reshape(n), token_ids.reshape(n),
                    emb, level_embed)
    return out.reshape(b, s, emb.shape[1])


# speedup vs baseline: 6.7754x; 6.7754x over previous
import functools

import jax
import jax.numpy as jnp
from jax import lax
from jax.experimental import pallas as pl
from jax.experimental.pallas import tpu as pltpu
from jax.experimental.pallas import tpu_sc as plsc

NC, NS = 2, 16
NW = NC * NS
C = 16
NB = 4
LOOK = NB - 2


def _sc_embed(level_ids_flat, token_ids_flat, emb, level_embed):
    n = token_ids_flat.shape[0]
    d = emb.shape[1]
    nlev = level_embed.shape[0]
    rpw = n // NW
    nchunk = rpw // C
    nround = nchunk // NB
    mesh = plsc.VectorSubcoreMesh(core_axis_name="c", subcore_axis_name="s",
                                  num_cores=NC, num_subcores=NS)

    @functools.partial(
        pl.kernel,
        out_type=jax.ShapeDtypeStruct((n, d), jnp.float32),
        mesh=mesh,
        scratch_types=[
            pltpu.VMEM((rpw,), jnp.int32),
            pltpu.VMEM((rpw,), jnp.int32),
            pltpu.VMEM((NB, C, d), jnp.float32),
            pltpu.VMEM((nlev, d), jnp.float32),
            pltpu.SemaphoreType.DMA((NB,)),
            pltpu.SemaphoreType.DMA((NB,)),
        ],
    )
    def k(lvl_hbm, tok_hbm, emb_hbm, lev_hbm, out_hbm,
          tok_v, lvl_v, rows, lev_v, sem_g, sem_s):
        wid = lax.axis_index("s") * NC + lax.axis_index("c")
        wbase = wid * rpw

        pltpu.sync_copy(tok_hbm.at[pl.ds(wbase, rpw)], tok_v)
        pltpu.sync_copy(lvl_hbm.at[pl.ds(wbase, rpw)], lvl_v)
        pltpu.sync_copy(lev_hbm, lev_v)

        def issue_gather(b, c):
            pltpu.async_copy(emb_hbm.at[tok_v.at[pl.ds(c * C, C)]],
                             rows.at[b], sem_g.at[b])

        def wait_gather(b):
            pltpu.make_async_copy(emb_hbm.at[tok_v.at[pl.ds(0, C)]],
                                  rows.at[b], sem_g.at[b]).wait()

        def issue_store(b, c):
            base = wbase + c * C
            pltpu.async_copy(rows.at[b], out_hbm.at[pl.ds(base, C)],
                             sem_s.at[b])

        def wait_store(b):
            pltpu.make_async_copy(rows.at[b], out_hbm.at[pl.ds(0, C)],
                                  sem_s.at[b]).wait()

        for c0 in range(LOOK):
            issue_gather(c0, c0)

        def round_body(g, carry):
            for b in range(NB):
                c = g * NB + b
                wait_gather(b)

                v = lvl_v[pl.ds(c * C, 16)]
                levs = [v[t] for t in range(16)]

                @plsc.parallel_loop(0, d // 16, unroll=2)
                def _(j, _b=b, _levs=levs):
                    sl = pl.ds(j * 16, 16)
                    for i in range(C):
                        plsc.addupdate(rows.at[_b, i, sl],
                                       lev_v[_levs[i], sl])

                issue_store(b, c)

                bn = (b + LOOK) % NB

                @pl.when(c + LOOK >= NB)
                def _():
                    wait_store(bn)

                @pl.when(c + LOOK < nchunk)
                def _():
                    issue_gather(bn, c + LOOK)
            return carry

        lax.fori_loop(0, nround, round_body, 0)
        for ct in range(nchunk - NB + LOOK, nchunk):
            wait_store(ct % NB)

    return k(level_ids_flat, token_ids_flat, emb, level_embed)


def kernel(level_ids, token_ids, emb, level_embed):
    b, s = token_ids.shape
    n = b * s
    out = _sc_embed(level_ids.reshape(n), token_ids.reshape(n),
                    emb, level_embed)
    return out.reshape(b, s, emb.shape[1])


# speedup vs baseline: 7.2596x; 1.0715x over previous
import functools

import jax
import jax.numpy as jnp
from jax import lax
from jax.experimental import pallas as pl
from jax.experimental.pallas import tpu as pltpu
from jax.experimental.pallas import tpu_sc as plsc

NC, NS = 2, 16
NW = NC * NS
C = 16
NB = 4
LOOK = NB - 2


def _sc_embed(level_ids_flat, token_ids_flat, emb, level_embed):
    n = token_ids_flat.shape[0]
    d = emb.shape[1]
    nlev = level_embed.shape[0]
    rpw = n // NW
    nchunk = rpw // C
    nround = nchunk // NB
    mesh = plsc.VectorSubcoreMesh(core_axis_name="c", subcore_axis_name="s",
                                  num_cores=NC, num_subcores=NS)

    @functools.partial(
        pl.kernel,
        out_type=jax.ShapeDtypeStruct((n, d), jnp.float32),
        mesh=mesh,
        scratch_types=[
            pltpu.VMEM((rpw,), jnp.int32),
            pltpu.VMEM((rpw,), jnp.int32),
            pltpu.VMEM((NB, C, d), jnp.float32),
            pltpu.VMEM((nlev, d), jnp.float32),
            pltpu.SemaphoreType.DMA((NB,)),
            pltpu.SemaphoreType.DMA((NB,)),
        ],
    )
    def k(lvl_hbm, tok_hbm, emb_hbm, lev_hbm, out_hbm,
          tok_v, lvl_v, rows, lev_v, sem_g, sem_s):
        wid = lax.axis_index("s") * NC + lax.axis_index("c")
        wbase = wid * rpw

        pltpu.sync_copy(tok_hbm.at[pl.ds(wbase, rpw)], tok_v)
        pltpu.sync_copy(lvl_hbm.at[pl.ds(wbase, rpw)], lvl_v)
        pltpu.sync_copy(lev_hbm, lev_v)

        def issue_gather(b, c):
            pltpu.async_copy(emb_hbm.at[tok_v.at[pl.ds(c * C, C)]],
                             rows.at[b], sem_g.at[b])

        def wait_gather(b):
            pltpu.make_async_copy(emb_hbm.at[tok_v.at[pl.ds(0, C)]],
                                  rows.at[b], sem_g.at[b]).wait()

        def issue_store(b, c):
            base = wbase + c * C
            pltpu.async_copy(rows.at[b], out_hbm.at[pl.ds(base, C)],
                             sem_s.at[b])

        def wait_store(b):
            pltpu.make_async_copy(rows.at[b], out_hbm.at[pl.ds(0, C)],
                                  sem_s.at[b]).wait()

        for c0 in range(LOOK):
            issue_gather(c0, c0)

        def round_body(g, carry):
            for b in range(NB):
                c = g * NB + b
                v = lvl_v[pl.ds(c * C, 16)]
                levs = [v[t] for t in range(16)]

                wait_gather(b)

                bn = (b + LOOK) % NB

                @pl.when(c + LOOK >= NB)
                def _():
                    wait_store(bn)

                @pl.when(c + LOOK < nchunk)
                def _():
                    issue_gather(bn, c + LOOK)

                @plsc.parallel_loop(0, d // 16, unroll=2)
                def _(j, _b=b, _levs=levs):
                    sl = pl.ds(j * 16, 16)
                    for i in range(C):
                        plsc.addupdate(rows.at[_b, i, sl],
                                       lev_v[_levs[i], sl])

                issue_store(b, c)
            return carry

        lax.fori_loop(0, nround, round_body, 0)
        for ct in range(nchunk - NB + LOOK, nchunk):
            wait_store(ct % NB)

    return k(level_ids_flat, token_ids_flat, emb, level_embed)


def kernel(level_ids, token_ids, emb, level_embed):
    b, s = token_ids.shape
    n = b * s
    out = _sc_embed(level_ids.reshape(n), token_ids.reshape(n),
                    emb, level_embed)
    return out.reshape(b, s, emb.shape[1])


# speedup vs baseline: 7.2890x; 1.0041x over previous
import functools

import jax
import jax.numpy as jnp
from jax import lax
from jax.experimental import pallas as pl
from jax.experimental.pallas import tpu as pltpu
from jax.experimental.pallas import tpu_sc as plsc

NC, NS = 2, 16
NW = NC * NS
C = 8
NB = 8
LOOK = NB - 2


def _sc_embed(level_ids_flat, token_ids_flat, emb, level_embed):
    n = token_ids_flat.shape[0]
    d = emb.shape[1]
    nlev = level_embed.shape[0]
    rpw = n // NW
    nchunk = rpw // C
    nround = nchunk // NB
    mesh = plsc.VectorSubcoreMesh(core_axis_name="c", subcore_axis_name="s",
                                  num_cores=NC, num_subcores=NS)

    @functools.partial(
        pl.kernel,
        out_type=jax.ShapeDtypeStruct((n, d), jnp.float32),
        mesh=mesh,
        scratch_types=[
            pltpu.VMEM((rpw,), jnp.int32),
            pltpu.VMEM((rpw + 8,), jnp.int32),
            pltpu.VMEM((NB, C, d), jnp.float32),
            pltpu.VMEM((nlev, d), jnp.float32),
            pltpu.SemaphoreType.DMA((NB,)),
            pltpu.SemaphoreType.DMA((NB,)),
        ],
    )
    def k(lvl_hbm, tok_hbm, emb_hbm, lev_hbm, out_hbm,
          tok_v, lvl_v, rows, lev_v, sem_g, sem_s):
        wid = lax.axis_index("s") * NC + lax.axis_index("c")
        wbase = wid * rpw

        pltpu.sync_copy(tok_hbm.at[pl.ds(wbase, rpw)], tok_v)
        pltpu.sync_copy(lvl_hbm.at[pl.ds(wbase, rpw)], lvl_v.at[pl.ds(0, rpw)])
        pltpu.sync_copy(lev_hbm, lev_v)

        def issue_gather(b, c):
            pltpu.async_copy(emb_hbm.at[tok_v.at[pl.ds(c * C, C)]],
                             rows.at[b], sem_g.at[b])

        def wait_gather(b):
            pltpu.make_async_copy(emb_hbm.at[tok_v.at[pl.ds(0, C)]],
                                  rows.at[b], sem_g.at[b]).wait()

        def issue_store(b, c):
            base = wbase + c * C
            pltpu.async_copy(rows.at[b], out_hbm.at[pl.ds(base, C)],
                             sem_s.at[b])

        def wait_store(b):
            pltpu.make_async_copy(rows.at[b], out_hbm.at[pl.ds(0, C)],
                                  sem_s.at[b]).wait()

        for c0 in range(LOOK):
            issue_gather(c0, c0)

        def round_body(g, carry):
            for b in range(NB):
                c = g * NB + b
                v = lvl_v[pl.ds(c * C, 16)]
                levs = [v[t] for t in range(C)]

                wait_gather(b)

                bn = (b + LOOK) % NB

                @pl.when(c + LOOK >= NB)
                def _():
                    wait_store(bn)

                @pl.when(c + LOOK < nchunk)
                def _():
                    issue_gather(bn, c + LOOK)

                @plsc.parallel_loop(0, d // 16, unroll=2)
                def _(j, _b=b, _levs=levs):
                    sl = pl.ds(j * 16, 16)
                    for i in range(C):
                        plsc.addupdate(rows.at[_b, i, sl],
                                       lev_v[_levs[i], sl])

                issue_store(b, c)
            return carry

        lax.fori_loop(0, nround, round_body, 0)
        for ct in range(nchunk - NB + LOOK, nchunk):
            wait_store(ct % NB)

    return k(level_ids_flat, token_ids_flat, emb, level_embed)


def kernel(level_ids, token_ids, emb, level_embed):
    b, s = token_ids.shape
    n = b * s
    out = _sc_embed(level_ids.reshape(n), token_ids.reshape(n),
                    emb, level_embed)
    return out.reshape(b, s, emb.shape[1])
